# Initial kernel scaffold; baseline (speedup 1.0000x reference)
import jax
import jax.numpy as jnp
from jax.experimental import pallas as pl


def kernel(edge_index, feat, W0, attn_l0, attn_r0, W1, attn_l1, attn_r1):
    raise NotImplementedError("write your pallas kernel here")


# speedup vs baseline: 31.8381x; 31.8381x over previous
import functools

import jax
import jax.numpy as jnp
from jax import lax
from jax.experimental import pallas as pl
from jax.experimental.pallas import tpu as pltpu
from jax.experimental.pallas import tpu_sc as plsc

N = 50000
E = 1600000
F = 1433
H0, D0 = 8, 8
H1, D1 = 1, 7
NC, NS = 2, 16
CB = 80
BR = 400

_MESH = plsc.VectorSubcoreMesh(core_axis_name="c", subcore_axis_name="s")
_SC_PARAMS = pltpu.CompilerParams(
    needs_layout_passes=False, use_tc_tiling_on_sc=False)


def _tc1_body(feat_ref, w0_ref, al_ref, ar_ref, h2_ref, elr_ref):
    h = jnp.dot(feat_ref[...], w0_ref[...], preferred_element_type=jnp.float32)
    h2_ref[0] = h[:, :32]
    h2_ref[1] = h[:, 32:]
    el = jnp.dot(h, al_ref[...], preferred_element_type=jnp.float32)
    er = jnp.dot(h, ar_ref[...], preferred_element_type=jnp.float32)
    elr_ref[...] = jnp.concatenate([el, er], axis=1)


def _tc1(feat, w0, al, ar):
    return pl.pallas_call(
        _tc1_body,
        grid=(N // BR,),
        in_specs=[
            pl.BlockSpec((BR, F), lambda i: (i, 0)),
            pl.BlockSpec((F, H0 * D0), lambda i: (0, 0)),
            pl.BlockSpec((H0 * D0, H0), lambda i: (0, 0)),
            pl.BlockSpec((H0 * D0, H0), lambda i: (0, 0)),
        ],
        out_specs=[
            pl.BlockSpec((2, BR, 32), lambda i: (0, i, 0)),
            pl.BlockSpec((BR, 16), lambda i: (i, 0)),
        ],
        out_shape=[
            jax.ShapeDtypeStruct((2, N, 32), jnp.float32),
            jax.ShapeDtypeStruct((N, 16), jnp.float32),
        ],
    )(feat, w0, al, ar)


def _sc1_body(src_ref, dst_ref, elr_ref, z16_ref, ex_ref, spart_ref,
              idx_s, idx_d, rows_s, rows_d, exbuf, sacc, sem1, sem2):
    c = lax.axis_index("c")
    s = lax.axis_index("s")
    wid = c * NS + s
    per_w = E // (NC * NS)
    nchunk = per_w // CB

    @pl.when(s == 0)
    def _():
        pltpu.sync_copy(z16_ref, sacc)

    plsc.subcore_barrier()

    lanes = lax.iota(jnp.int32, 16)
    perm = (lanes + 8) % 16

    def chunk(i, carry):
        base = wid * per_w + i * CB
        pltpu.sync_copy(src_ref.at[pl.ds(base, CB)], idx_s)
        pltpu.sync_copy(dst_ref.at[pl.ds(base, CB)], idx_d)
        pltpu.async_copy(elr_ref.at[idx_s], rows_s, sem1).wait()
        pltpu.async_copy(elr_ref.at[idx_d], rows_d, sem2).wait()

        def edge(e, carry2):
            ef = jnp.full((16,), e, jnp.int32)
            vs = rows_s[e, :]
            vrot = plsc.load_gather(rows_d, [ef, perm])
            v = vs + vrot
            v = jnp.where(v > 0, v, 0.2 * v)
            exbuf[e, :] = jnp.exp(v)
            return carry2

        lax.fori_loop(0, CB, edge, 0)
        pltpu.sync_copy(exbuf, sacc.at[idx_d], add=True)
        pltpu.sync_copy(exbuf, ex_ref.at[pl.ds(base, CB)])
        return carry

    lax.fori_loop(0, nchunk, chunk, 0)
    plsc.subcore_barrier()

    @pl.when(s == 0)
    def _():
        pltpu.sync_copy(sacc, spart_ref.at[c])


def _sc1(src, dst, elr, z16):
    f = pl.kernel(
        _sc1_body,
        out_type=[
            jax.ShapeDtypeStruct((E, 16), jnp.float32),
            jax.ShapeDtypeStruct((2, N, 16), jnp.float32),
        ],
        mesh=_MESH,
        compiler_params=_SC_PARAMS,
        scratch_types=[
            pltpu.VMEM((CB,), jnp.int32),
            pltpu.VMEM((CB,), jnp.int32),
            pltpu.VMEM((CB, 16), jnp.float32),
            pltpu.VMEM((CB, 16), jnp.float32),
            pltpu.VMEM((CB, 16), jnp.float32),
            pltpu.VMEM_SHARED((N, 16), jnp.float32),
            pltpu.SemaphoreType.DMA,
            pltpu.SemaphoreType.DMA,
        ],
    )
    return f(src, dst, elr, z16)


def _sc2_body(src_ref, dst_ref, h2_ref, ex_ref, z32_ref, mpart_ref,
              idx_s, idx_g, idx_d, hrows, exbuf, msgbuf, macc, sem1):
    c = lax.axis_index("c")
    s = lax.axis_index("s")
    per_s = E // NS
    nchunk = per_s // CB

    @pl.when(s == 0)
    def _():
        pltpu.sync_copy(z32_ref, macc)

    plsc.subcore_barrier()

    heads = lax.iota(jnp.int32, 16) // 8
    col0 = heads + 4 * c
    col1 = col0 + 2
    row_off = c * N

    def chunk(i, carry):
        base = s * per_s + i * CB
        pltpu.sync_copy(src_ref.at[pl.ds(base, CB)], idx_s)
        pltpu.sync_copy(dst_ref.at[pl.ds(base, CB)], idx_d)
        for j in range(CB // 16):
            idx_g[pl.ds(j * 16, 16)] = idx_s[pl.ds(j * 16, 16)] + row_off
        pltpu.async_copy(h2_ref.at[idx_g], hrows, sem1).wait()
        pltpu.sync_copy(ex_ref.at[pl.ds(base, CB)], exbuf)

        def edge(e, carry2):
            ef = jnp.full((16,), e, jnp.int32)
            ex0 = plsc.load_gather(exbuf, [ef, col0])
            ex1 = plsc.load_gather(exbuf, [ef, col1])
            msgbuf[e, pl.ds(0, 16)] = hrows[e, pl.ds(0, 16)] * ex0
            msgbuf[e, pl.ds(16, 16)] = hrows[e, pl.ds(16, 16)] * ex1
            return carry2

        lax.fori_loop(0, CB, edge, 0)
        pltpu.sync_copy(msgbuf, macc.at[idx_d], add=True)
        return carry

    lax.fori_loop(0, nchunk, chunk, 0)
    plsc.subcore_barrier()

    @pl.when(s == 0)
    def _():
        pltpu.sync_copy(macc, mpart_ref.at[c])


def _sc2(src, dst, h2f, ex, z32):
    f = pl.kernel(
        _sc2_body,
        out_type=jax.ShapeDtypeStruct((2, N, 32), jnp.float32),
        mesh=_MESH,
        compiler_params=_SC_PARAMS,
        scratch_types=[
            pltpu.VMEM((CB,), jnp.int32),
            pltpu.VMEM((CB,), jnp.int32),
            pltpu.VMEM((CB,), jnp.int32),
            pltpu.VMEM((CB, 32), jnp.float32),
            pltpu.VMEM((CB, 16), jnp.float32),
            pltpu.VMEM((CB, 32), jnp.float32),
            pltpu.VMEM_SHARED((N, 32), jnp.float32),
            pltpu.SemaphoreType.DMA,
        ],
    )
    return f(src, dst, h2f, ex, z32)


def _tc2_body(mp_ref, sp_ref, w1_ref, al1_ref, ar1_ref, r_ref, t1_ref):
    x = jnp.concatenate([mp_ref[0], mp_ref[1]], axis=1)
    s8 = sp_ref[0, :, 0:8] + sp_ref[1, :, 0:8]
    srep = jnp.dot(s8, r_ref[...], preferred_element_type=jnp.float32)
    out0 = x / (srep + 1e-9)
    out0 = jnp.where(out0 > 0, out0, jnp.exp(out0) - 1.0)
    h1 = jnp.dot(out0, w1_ref[...], preferred_element_type=jnp.float32)
    el1 = jnp.sum(h1 * al1_ref[...], axis=1, keepdims=True)
    er1 = jnp.sum(h1 * ar1_ref[...], axis=1, keepdims=True)
    pad = jnp.zeros((h1.shape[0], 7), jnp.float32)
    t1_ref[...] = jnp.concatenate([h1, el1, er1, pad], axis=1)


def _tc2(m_part, s_part, w1, al1, ar1, r):
    return pl.pallas_call(
        _tc2_body,
        grid=(N // BR,),
        in_specs=[
            pl.BlockSpec((2, BR, 32), lambda i: (0, i, 0)),
            pl.BlockSpec((2, BR, 16), lambda i: (0, i, 0)),
            pl.BlockSpec((H0 * D0, H1 * D1), lambda i: (0, 0)),
            pl.BlockSpec((H1, D1), lambda i: (0, 0)),
            pl.BlockSpec((H1, D1), lambda i: (0, 0)),
            pl.BlockSpec((H0, H0 * D0), lambda i: (0, 0)),
        ],
        out_specs=pl.BlockSpec((BR, 16), lambda i: (i, 0)),
        out_shape=jax.ShapeDtypeStruct((N, 16), jnp.float32),
    )(m_part, s_part, w1, al1, ar1, r)


def _sc3_body(src_ref, dst_ref, t1_ref, z16_ref, apart_ref,
              idx_s, idx_d, rows_s, rows_d, msgbuf, aacc, sem1, sem2):
    c = lax.axis_index("c")
    s = lax.axis_index("s")
    wid = c * NS + s
    per_w = E // (NC * NS)
    nchunk = per_w // CB

    @pl.when(s == 0)
    def _():
        pltpu.sync_copy(z16_ref, aacc)

    plsc.subcore_barrier()

    lanes = lax.iota(jnp.int32, 16)
    c7 = jnp.full((16,), 7, jnp.int32)
    c8 = jnp.full((16,), 8, jnp.int32)

    def chunk(i, carry):
        base = wid * per_w + i * CB
        pltpu.sync_copy(src_ref.at[pl.ds(base, CB)], idx_s)
        pltpu.sync_copy(dst_ref.at[pl.ds(base, CB)], idx_d)
        pltpu.async_copy(t1_ref.at[idx_s], rows_s, sem1).wait()
        pltpu.async_copy(t1_ref.at[idx_d], rows_d, sem2).wait()

        def edge(e, carry2):
            ef = jnp.full((16,), e, jnp.int32)
            vs = rows_s[e, :]
            elb = plsc.load_gather(rows_s, [ef, c7])
            erb = plsc.load_gather(rows_d, [ef, c8])
            v = elb + erb
            v = jnp.where(v > 0, v, 0.2 * v)
            exv = jnp.exp(v)
            contrib = jnp.where(lanes < 7, vs * exv,
                                jnp.where(lanes == 7, exv, 0.0))
            msgbuf[e, :] = contrib
            return carry2

        lax.fori_loop(0, CB, edge, 0)
        pltpu.sync_copy(msgbuf, aacc.at[idx_d], add=True)
        return carry

    lax.fori_loop(0, nchunk, chunk, 0)
    plsc.subcore_barrier()

    @pl.when(s == 0)
    def _():
        pltpu.sync_copy(aacc, apart_ref.at[c])


def _sc3(src, dst, t1, z16):
    f = pl.kernel(
        _sc3_body,
        out_type=jax.ShapeDtypeStruct((2, N, 16), jnp.float32),
        mesh=_MESH,
        compiler_params=_SC_PARAMS,
        scratch_types=[
            pltpu.VMEM((CB,), jnp.int32),
            pltpu.VMEM((CB,), jnp.int32),
            pltpu.VMEM((CB, 16), jnp.float32),
            pltpu.VMEM((CB, 16), jnp.float32),
            pltpu.VMEM((CB, 16), jnp.float32),
            pltpu.VMEM_SHARED((N, 16), jnp.float32),
            pltpu.SemaphoreType.DMA,
            pltpu.SemaphoreType.DMA,
        ],
    )
    return f(src, dst, t1, z16)


def _tc3_body(ap_ref, out_ref):
    a = ap_ref[0] + ap_ref[1]
    out_ref[...] = a[:, 0:7] / (a[:, 7:8] + 1e-9)


def _tc3(a_part):
    return pl.pallas_call(
        _tc3_body,
        grid=(N // BR,),
        in_specs=[pl.BlockSpec((2, BR, 16), lambda i: (0, i, 0))],
        out_specs=pl.BlockSpec((BR, D1), lambda i: (i, 0)),
        out_shape=jax.ShapeDtypeStruct((N, D1), jnp.float32),
    )(a_part)


def kernel(edge_index, feat, W0, attn_l0, attn_r0, W1, attn_l1, attn_r1):
    src = edge_index[0].astype(jnp.int32)
    dst = edge_index[1].astype(jnp.int32)

    eye = jnp.eye(H0, dtype=jnp.float32)
    a_l = (attn_l0[:, :, None] * eye[:, None, :]).reshape(H0 * D0, H0)
    a_r = (attn_r0[:, :, None] * eye[:, None, :]).reshape(H0 * D0, H0)
    rrep = jnp.kron(eye, jnp.ones((1, D0), jnp.float32))

    z16 = jnp.zeros((N, 16), jnp.float32)
    z32 = jnp.zeros((N, 32), jnp.float32)

    h2, elr = _tc1(feat, W0, a_l, a_r)
    h2f = h2.reshape(2 * N, 32)

    ex, s_part = _sc1(src, dst, elr, z16)
    m_part = _sc2(src, dst, h2f, ex, z32)
    t1 = _tc2(m_part, s_part, W1, attn_l1, attn_r1, rrep)
    a_part = _sc3(src, dst, t1, z16)
    return _tc3(a_part)
